# baseline (speedup 1.0000x reference)
import functools

import jax
import jax.numpy as jnp
from jax.experimental import pallas as pl
from jax.experimental.pallas import tpu as pltpu

NUM_ENT = 50000
NUM_REL = 474
D = 128
NODE_D = 32
B = 128
NB = 10
THRESH = 1373

VOCAB_CHUNK = 2048


def _dense_body(h_ref, e_ref, cnt_ref, r_ref, nod_ref, wve_ref, wvn_ref,
                bv_ref, f1a_ref, f1b_ref, b1_ref, f2_ref, b2_ref, ent_ref,
                yc_ref, out_s):
    @pl.when(pl.program_id(0) == 0)
    def _():
        cnt = cnt_ref[...]
        node = (NB - cnt) * nod_ref[0:1, :] + cnt * nod_ref[1:2, :]
        V = (h_ref[...]
             + jnp.dot(e_ref[...], wve_ref[...], preferred_element_type=jnp.float32)
             + jnp.dot(node, wvn_ref[...], preferred_element_type=jnp.float32)
             + NB * bv_ref[...])
        z1 = jnp.maximum(
            jnp.dot(V, f1a_ref[...], preferred_element_type=jnp.float32)
            + jnp.dot(r_ref[...], f1b_ref[...], preferred_element_type=jnp.float32)
            + b1_ref[...], 0.0)
        out_s[...] = (jnp.dot(z1, f2_ref[...], preferred_element_type=jnp.float32)
                      + b2_ref[...])

    logits = jax.lax.dot_general(out_s[...], ent_ref[...],
                                 (((1,), (1,)), ((), ())),
                                 preferred_element_type=jnp.float32)
    yc_ref[...] = jax.nn.sigmoid(logits)


def _dense_stage(h_emb, e_sum, cnt_f, r_emb, nod_embed, WV, bV,
                 fc1_w, fc1_b, fc2_w, fc2_b, ent_embed):
    n_chunks = pl.cdiv(NUM_ENT, VOCAB_CHUNK)
    const = lambda shape: pl.BlockSpec(shape, lambda i: (0, 0))
    return pl.pallas_call(
        _dense_body,
        grid=(n_chunks,),
        in_specs=[
            const((B, D)),
            const((B, D)),
            const((B, 1)),
            const((B, D)),
            const((2, NODE_D)),
            const((D, D)),
            const((NODE_D, D)),
            const((1, D)),
            const((D, D)),
            const((D, D)),
            const((1, D)),
            const((D, D)),
            const((1, D)),
            pl.BlockSpec((VOCAB_CHUNK, D), lambda i: (i, 0)),
        ],
        out_specs=pl.BlockSpec((B, VOCAB_CHUNK), lambda i: (0, i)),
        out_shape=jax.ShapeDtypeStruct((B, NUM_ENT), jnp.float32),
        scratch_shapes=[pltpu.VMEM((B, D), jnp.float32)],
    )(h_emb, e_sum, cnt_f, r_emb, nod_embed,
      WV[:, :D].T, WV[:, D:].T, bV.reshape(1, D),
      fc1_w[:, :D].T, fc1_w[:, D:].T, fc1_b.reshape(1, D),
      fc2_w.T, fc2_b.reshape(1, D), ent_embed)


def kernel(src, rel, t_idxs, ent_embed, rel_embed, nod_embed,
           WQ, bQ, WK, bK, WV, bV, fc1_w, fc1_b, fc2_w, fc2_b):
    nbrs = jnp.take(t_idxs, src, axis=0)
    h_emb = jnp.take(ent_embed, src, axis=0)
    e_sum = jnp.sum(jnp.take(ent_embed, nbrs, axis=0), axis=1)
    cnt_f = jnp.sum((nbrs >= THRESH).astype(jnp.float32), axis=1).reshape(B, 1)
    r_emb = jnp.take(rel_embed, rel, axis=0)
    return _dense_stage(h_emb, e_sum, cnt_f, r_emb, nod_embed, WV, bV,
                        fc1_w, fc1_b, fc2_w, fc2_b, ent_embed)
